# Initial kernel scaffold; baseline (speedup 1.0000x reference)
import jax
import jax.numpy as jnp
from jax.experimental import pallas as pl


def kernel(x, edge_index):
    raise NotImplementedError("write your pallas kernel here")


# speedup vs baseline: 15.8316x; 15.8316x over previous
import jax
import jax.numpy as jnp
from jax import lax
from jax.experimental import pallas as pl
from jax.experimental.pallas import tpu as pltpu
from jax.experimental.pallas import tpu_sc as plsc

N = 10000
D = 128
H = D // 2
E = 320000
ALPHA = 0.1
CK = 0.45

NC = 2
NS = 16
NW = NC * NS
C = 128
KB = 4
NCH = 80
E_PAD = NW * NCH * C
N_PAD = 10240
RPT = N_PAD // NS

_f32 = jnp.float32
_mesh = plsc.VectorSubcoreMesh(core_axis_name="c", subcore_axis_name="s")
_sc_params = pltpu.CompilerParams(use_tc_tiling_on_sc=False)


def _hist_body(coli_hbm, z16_hbm, out_hbm, colidx_v, ones_v, acc_sh, *sems):
    c = lax.axis_index("c")
    s = lax.axis_index("s")
    w = c * NS + s
    pltpu.sync_copy(coli_hbm.at[w], colidx_v)

    def fill(i, carry):
        ones_v[i, :] = jnp.ones((16,), _f32)
        return carry

    lax.fori_loop(0, C, fill, 0)
    pltpu.sync_copy(z16_hbm.at[pl.ds(s * RPT, RPT)],
                    acc_sh.at[pl.ds(s * RPT, RPT)])
    plsc.subcore_barrier()

    def phase(p, carry):
        descs = [
            pltpu.async_copy(ones_v, acc_sh.at[colidx_v.at[p * KB + j]],
                             sems[j], add=True)
            for j in range(KB)
        ]
        for d in descs:
            d.wait()
        return carry

    lax.fori_loop(0, NCH // KB, phase, 0)
    plsc.subcore_barrier()
    pltpu.sync_copy(acc_sh.at[pl.ds(s * RPT, RPT)],
                    out_hbm.at[c, pl.ds(s * RPT, RPT)])


_hist_kernel = pl.kernel(
    _hist_body,
    out_type=jax.ShapeDtypeStruct((NC, N_PAD, 16), _f32),
    mesh=_mesh,
    scratch_types=[
        pltpu.VMEM((NCH, C), jnp.int32),
        pltpu.VMEM((C, 16), _f32),
        pltpu.VMEM_SHARED((N_PAD, 16), _f32),
    ] + [pltpu.SemaphoreType.DMA] * KB,
    compiler_params=_sc_params,
)


def _scatter_body(g_hbm, rowi_hbm, coli_hbm, z_hbm, out_hbm,
                  rowidx_v, colidx_v, b0, b1, b2, b3, acc_sh, *sems):
    bufs = (b0, b1, b2, b3)
    gsems = sems[:KB]
    ssems = sems[KB:]
    c = lax.axis_index("c")
    s = lax.axis_index("s")
    w = c * NS + s
    pltpu.sync_copy(rowi_hbm.at[w], rowidx_v)
    pltpu.sync_copy(coli_hbm.at[w], colidx_v)
    pltpu.sync_copy(z_hbm.at[pl.ds(s * RPT, RPT)],
                    acc_sh.at[pl.ds(s * RPT, RPT)])
    plsc.subcore_barrier()

    def phase(p, carry):
        gd = [
            pltpu.async_copy(g_hbm.at[rowidx_v.at[p * KB + j]], bufs[j],
                             gsems[j])
            for j in range(KB)
        ]
        for d in gd:
            d.wait()
        sd = [
            pltpu.async_copy(bufs[j], acc_sh.at[colidx_v.at[p * KB + j]],
                             ssems[j], add=True)
            for j in range(KB)
        ]
        for d in sd:
            d.wait()
        return carry

    lax.fori_loop(0, NCH // KB, phase, 0)
    plsc.subcore_barrier()
    pltpu.sync_copy(acc_sh.at[pl.ds(s * RPT, RPT)],
                    out_hbm.at[c, pl.ds(s * RPT, RPT)])


_scatter_kernel = pl.kernel(
    _scatter_body,
    out_type=jax.ShapeDtypeStruct((NC, N_PAD, H), _f32),
    mesh=_mesh,
    scratch_types=[
        pltpu.VMEM((NCH, C), jnp.int32),
        pltpu.VMEM((NCH, C), jnp.int32),
        pltpu.VMEM((C, H), _f32),
        pltpu.VMEM((C, H), _f32),
        pltpu.VMEM((C, H), _f32),
        pltpu.VMEM((C, H), _f32),
        pltpu.VMEM_SHARED((N_PAD, H), _f32),
    ] + [pltpu.SemaphoreType.DMA] * (2 * KB),
    compiler_params=_sc_params,
)


_RB = 1024
_GRID = N_PAD // _RB

_half_spec = pl.BlockSpec((_RB, H), lambda i: (i, 0))
_one_spec = pl.BlockSpec((_RB, 1), lambda i: (i, 0))
_pair_spec = pl.BlockSpec((NC, _RB, H), lambda i: (0, i, 0))
_half_shape = jax.ShapeDtypeStruct((N_PAD, H), _f32)
_one_shape = jax.ShapeDtypeStruct((N_PAD, 1), _f32)


def _norm_body(hist_ref, xl_ref, xh_ref, dis_ref, invc_ref, g0l_ref, g0h_ref):
    cnt = hist_ref[0, :, 0:1] + hist_ref[1, :, 0:1]
    dis = lax.rsqrt(cnt + 1.0)
    dis_ref[...] = dis
    invc_ref[...] = 1.0 / jnp.maximum(cnt, 1.0)
    g0l_ref[...] = dis * xl_ref[...]
    g0h_ref[...] = dis * xh_ref[...]


_norm_kernel = pl.pallas_call(
    _norm_body,
    grid=(_GRID,),
    in_specs=[
        pl.BlockSpec((NC, _RB, 16), lambda i: (0, i, 0)),
        _half_spec, _half_spec,
    ],
    out_specs=[_one_spec, _one_spec, _half_spec, _half_spec],
    out_shape=[_one_shape, _one_shape, _half_shape, _half_shape],
)


def _round1_body(pl_ref, ph_ref, g0l_ref, g0h_ref, xl_ref, xh_ref, dis_ref,
                 g1l_ref, g1h_ref, o1l_ref, o1h_ref):
    dis = dis_ref[...]
    h1l = dis * (pl_ref[0] + pl_ref[1] + g0l_ref[...])
    h1h = dis * (ph_ref[0] + ph_ref[1] + g0h_ref[...])
    g1l_ref[...] = dis * h1l
    g1h_ref[...] = dis * h1h
    o1l_ref[...] = ALPHA * xl_ref[...] + CK * h1l
    o1h_ref[...] = ALPHA * xh_ref[...] + CK * h1h


_round1_kernel = pl.pallas_call(
    _round1_body,
    grid=(_GRID,),
    in_specs=[_pair_spec, _pair_spec, _half_spec, _half_spec,
              _half_spec, _half_spec, _one_spec],
    out_specs=[_half_spec, _half_spec, _half_spec, _half_spec],
    out_shape=[_half_shape, _half_shape, _half_shape, _half_shape],
)


def _round2_body(ql_ref, qh_ref, g1l_ref, g1h_ref, o1l_ref, o1h_ref, dis_ref,
                 rl_ref, rh_ref):
    dis = dis_ref[...]
    h2l = dis * (ql_ref[0] + ql_ref[1] + g1l_ref[...])
    h2h = dis * (qh_ref[0] + qh_ref[1] + g1h_ref[...])
    rl_ref[...] = jnp.maximum(o1l_ref[...] + CK * h2l, 0.0)
    rh_ref[...] = jnp.maximum(o1h_ref[...] + CK * h2h, 0.0)


_round2_kernel = pl.pallas_call(
    _round2_body,
    grid=(_GRID,),
    in_specs=[_pair_spec, _pair_spec, _half_spec, _half_spec,
              _half_spec, _half_spec, _one_spec],
    out_specs=[_half_spec, _half_spec],
    out_shape=[_half_shape, _half_shape],
)


def _final_body(pl_ref, ph_ref, xl_ref, xh_ref, invc_ref, ol_ref, oh_ref):
    invc = invc_ref[...]
    ol_ref[...] = jnp.maximum((pl_ref[0] + pl_ref[1]) * invc + xl_ref[...], 0.0)
    oh_ref[...] = jnp.maximum((ph_ref[0] + ph_ref[1]) * invc + xh_ref[...], 0.0)


_final_kernel = pl.pallas_call(
    _final_body,
    grid=(_GRID,),
    in_specs=[_pair_spec, _pair_spec, _half_spec, _half_spec, _one_spec],
    out_specs=[_half_spec, _half_spec],
    out_shape=[_half_shape, _half_shape],
)


@jax.jit
def kernel(x, edge_index):
    row = edge_index[0]
    col = edge_index[1]
    fill = (jnp.arange(E_PAD - E, dtype=jnp.int32) % (N_PAD - N)) + N
    row_p = jnp.concatenate([row, fill]).reshape(NW, NCH, C)
    col_p = jnp.concatenate([col, fill]).reshape(NW, NCH, C)
    x_p = jnp.pad(x, ((0, N_PAD - N), (0, 0)))
    x_l, x_h = x_p[:, :H], x_p[:, H:]
    z16 = jnp.zeros((N_PAD, 16), _f32)
    zh = jnp.zeros((N_PAD, H), _f32)

    hist = _hist_kernel(col_p, z16)
    dis, invc, g0l, g0h = _norm_kernel(hist, x_l, x_h)
    p1l = _scatter_kernel(g0l, row_p, col_p, zh)
    p1h = _scatter_kernel(g0h, row_p, col_p, zh)
    g1l, g1h, o1l, o1h = _round1_kernel(p1l, p1h, g0l, g0h, x_l, x_h, dis)
    q2l = _scatter_kernel(g1l, row_p, col_p, zh)
    q2h = _scatter_kernel(g1h, row_p, col_p, zh)
    rl, rh = _round2_kernel(q2l, q2h, g1l, g1h, o1l, o1h, dis)
    p3l = _scatter_kernel(rl, row_p, col_p, zh)
    p3h = _scatter_kernel(rh, row_p, col_p, zh)
    outl, outh = _final_kernel(p3l, p3h, x_l, x_h, invc)
    return jnp.concatenate([outl[:N], outh[:N]], axis=1)


# speedup vs baseline: 19.3124x; 1.2199x over previous
import jax
import jax.numpy as jnp
from jax import lax
from jax.experimental import pallas as pl
from jax.experimental.pallas import tpu as pltpu
from jax.experimental.pallas import tpu_sc as plsc

N = 10000
D = 128
H = D // 2
E = 320000
ALPHA = 0.1
CK = 0.45

NC = 2
NS = 16
C = 128
KB = 2
NCH = 160
NPH = NCH // KB
E_PAD = NS * NCH * C
N_PAD = 10240
RPT = N_PAD // NS

_f32 = jnp.float32
_mesh = plsc.VectorSubcoreMesh(core_axis_name="c", subcore_axis_name="s")
_sc_params = pltpu.CompilerParams(use_tc_tiling_on_sc=False)


def _hist_body(coli_hbm, z16_hbm, out_hbm, colidx_v, ones_v, acc_sh, *sems):
    c = lax.axis_index("c")
    s = lax.axis_index("s")
    w = c * NS + s
    pltpu.sync_copy(coli_hbm.at[w], colidx_v)

    def fill(i, carry):
        ones_v[i, :] = jnp.ones((16,), _f32)
        return carry

    lax.fori_loop(0, C, fill, 0)
    pltpu.sync_copy(z16_hbm.at[pl.ds(s * RPT, RPT)],
                    acc_sh.at[pl.ds(s * RPT, RPT)])
    plsc.subcore_barrier()

    def phase(p, carry):
        descs = [
            pltpu.async_copy(ones_v, acc_sh.at[colidx_v.at[p * KB + j]],
                             sems[j], add=True)
            for j in range(KB)
        ]
        for d in descs:
            d.wait()
        return carry

    lax.fori_loop(0, (NCH // NC) // KB, phase, 0)
    plsc.subcore_barrier()
    pltpu.sync_copy(acc_sh.at[pl.ds(s * RPT, RPT)],
                    out_hbm.at[c, pl.ds(s * RPT, RPT)])


_hist_kernel = pl.kernel(
    _hist_body,
    out_type=jax.ShapeDtypeStruct((NC, N_PAD, 16), _f32),
    mesh=_mesh,
    scratch_types=[
        pltpu.VMEM((NCH // NC, C), jnp.int32),
        pltpu.VMEM((C, 16), _f32),
        pltpu.VMEM_SHARED((N_PAD, 16), _f32),
    ] + [pltpu.SemaphoreType.DMA] * KB,
    compiler_params=_sc_params,
)


def _scatter_body(g_hbm, init_hbm, rowi_hbm, coli_hbm, out_hbm,
                  rowidx_v, colidx_v,
                  a0, a1, b0, b1,
                  acc_sh, gsa, ssa, gsb, ssb):
    abufs = (a0, a1)
    bbufs = (b0, b1)
    c = lax.axis_index("c")
    s = lax.axis_index("s")
    pltpu.sync_copy(rowi_hbm.at[c, s], rowidx_v)
    pltpu.sync_copy(coli_hbm.at[s], colidx_v)
    pltpu.sync_copy(init_hbm.at[pl.ds(c * N_PAD + s * RPT, RPT)],
                    acc_sh.at[pl.ds(s * RPT, RPT)])
    plsc.subcore_barrier()

    def gathers(p, bufs, sem):
        return [
            pltpu.async_copy(g_hbm.at[rowidx_v.at[p * KB + j]], bufs[j], sem)
            for j in range(KB)
        ]

    def scatters(p, bufs, sem):
        return [
            pltpu.async_copy(bufs[j], acc_sh.at[colidx_v.at[p * KB + j]],
                             sem, add=True)
            for j in range(KB)
        ]

    def wait_all(descs):
        for d in descs:
            d.wait()

    wait_all(gathers(0, abufs, gsa))

    def pair(q, carry):
        pa = 2 * q
        pb = 2 * q + 1
        sa = scatters(pa, abufs, ssa)
        gb = gathers(pb, bbufs, gsb)
        wait_all(sa)
        wait_all(gb)
        sb = scatters(pb, bbufs, ssb)
        ga = gathers(pa + 2, abufs, gsa)
        wait_all(sb)
        wait_all(ga)
        return carry

    lax.fori_loop(0, NPH // 2 - 1, pair, 0)
    sa = scatters(NPH - 2, abufs, ssa)
    gb = gathers(NPH - 1, bbufs, gsb)
    wait_all(sa)
    wait_all(gb)
    wait_all(scatters(NPH - 1, bbufs, ssb))

    plsc.subcore_barrier()
    pltpu.sync_copy(acc_sh.at[pl.ds(s * RPT, RPT)],
                    out_hbm.at[c, pl.ds(s * RPT, RPT)])


_scatter_kernel = pl.kernel(
    _scatter_body,
    out_type=jax.ShapeDtypeStruct((NC, N_PAD, H), _f32),
    mesh=_mesh,
    scratch_types=[
        pltpu.VMEM((NCH, C), jnp.int32),
        pltpu.VMEM((NCH, C), jnp.int32),
    ] + [pltpu.VMEM((C, H), _f32)] * (2 * KB) + [
        pltpu.VMEM_SHARED((N_PAD, H), _f32),
    ] + [pltpu.SemaphoreType.DMA] * 4,
    compiler_params=_sc_params,
)


_RB = 1024
_GRID = N_PAD // _RB

_half_spec = pl.BlockSpec((_RB, H), lambda i: (i, 0))
_one_spec = pl.BlockSpec((_RB, 1), lambda i: (i, 0))
_pair_spec = pl.BlockSpec((NC, _RB, H), lambda i: (0, i, 0))
_half_shape = jax.ShapeDtypeStruct((N_PAD, H), _f32)
_one_shape = jax.ShapeDtypeStruct((N_PAD, 1), _f32)
_pair_shape = jax.ShapeDtypeStruct((NC, N_PAD, H), _f32)


def _norm_body(hist_ref, xl_ref, xh_ref, dis_ref, invc_ref, g0_ref):
    cnt = hist_ref[0, :, 0:1] + hist_ref[1, :, 0:1]
    dis = lax.rsqrt(cnt + 1.0)
    dis_ref[...] = dis
    invc_ref[...] = 1.0 / jnp.maximum(cnt, 1.0)
    g0_ref[0] = dis * xl_ref[...]
    g0_ref[1] = dis * xh_ref[...]


_norm_kernel = pl.pallas_call(
    _norm_body,
    grid=(_GRID,),
    in_specs=[
        pl.BlockSpec((NC, _RB, 16), lambda i: (0, i, 0)),
        _half_spec, _half_spec,
    ],
    out_specs=[_one_spec, _one_spec, _pair_spec],
    out_shape=[_one_shape, _one_shape, _pair_shape],
)


def _round1_body(p_ref, xl_ref, xh_ref, dis_ref, g1_ref, o1_ref):
    dis = dis_ref[...]
    h1l = dis * p_ref[0]
    h1h = dis * p_ref[1]
    g1_ref[0] = dis * h1l
    g1_ref[1] = dis * h1h
    o1_ref[0] = ALPHA * xl_ref[...] + CK * h1l
    o1_ref[1] = ALPHA * xh_ref[...] + CK * h1h


_round1_kernel = pl.pallas_call(
    _round1_body,
    grid=(_GRID,),
    in_specs=[_pair_spec, _half_spec, _half_spec, _one_spec],
    out_specs=[_pair_spec, _pair_spec],
    out_shape=[_pair_shape, _pair_shape],
)


def _round2_body(q_ref, o1_ref, dis_ref, r_ref):
    dis = dis_ref[...]
    r_ref[0] = jnp.maximum(o1_ref[0] + CK * dis * q_ref[0], 0.0)
    r_ref[1] = jnp.maximum(o1_ref[1] + CK * dis * q_ref[1], 0.0)


_round2_kernel = pl.pallas_call(
    _round2_body,
    grid=(_GRID,),
    in_specs=[_pair_spec, _pair_spec, _one_spec],
    out_specs=_pair_spec,
    out_shape=_pair_shape,
)


def _final_body(p_ref, xl_ref, xh_ref, invc_ref, ol_ref, oh_ref):
    invc = invc_ref[...]
    ol_ref[...] = jnp.maximum(p_ref[0] * invc + xl_ref[...], 0.0)
    oh_ref[...] = jnp.maximum(p_ref[1] * invc + xh_ref[...], 0.0)


_final_kernel = pl.pallas_call(
    _final_body,
    grid=(_GRID,),
    in_specs=[_pair_spec, _half_spec, _half_spec, _one_spec],
    out_specs=[_half_spec, _half_spec],
    out_shape=[_half_shape, _half_shape],
)


@jax.jit
def kernel(x, edge_index):
    row = edge_index[0]
    col = edge_index[1]
    fill = (jnp.arange(E_PAD - E, dtype=jnp.int32) % (N_PAD - N)) + N
    row_flat = jnp.concatenate([row, fill])
    row_p = jnp.stack([row_flat, row_flat + N_PAD]).reshape(NC, NS, NCH, C)
    col_p = jnp.concatenate([col, fill]).reshape(NS, NCH, C)
    x_p = jnp.pad(x, ((0, N_PAD - N), (0, 0)))
    x_l, x_h = x_p[:, :H], x_p[:, H:]
    z16 = jnp.zeros((N_PAD, 16), _f32)
    zpair = jnp.zeros((NC * N_PAD, H), _f32)

    hist = _hist_kernel(col_p.reshape(NC * NS, NCH // NC, C), z16)
    dis, invc, g0 = _norm_kernel(hist, x_l, x_h)
    g0f = g0.reshape(NC * N_PAD, H)
    p1 = _scatter_kernel(g0f, g0f, row_p, col_p)
    g1, o1 = _round1_kernel(p1, x_l, x_h, dis)
    g1f = g1.reshape(NC * N_PAD, H)
    q2 = _scatter_kernel(g1f, g1f, row_p, col_p)
    r = _round2_kernel(q2, o1, dis)
    rf = r.reshape(NC * N_PAD, H)
    p3 = _scatter_kernel(rf, zpair, row_p, col_p)
    outl, outh = _final_kernel(p3, x_l, x_h, invc)
    return jnp.concatenate([outl[:N], outh[:N]], axis=1)


# speedup vs baseline: 20.4838x; 1.0607x over previous
import jax
import jax.numpy as jnp
from jax import lax
from jax.experimental import pallas as pl
from jax.experimental.pallas import tpu as pltpu
from jax.experimental.pallas import tpu_sc as plsc

N = 10000
D = 128
H = D // 2
E = 320000
ALPHA = 0.1
CK = 0.45

NC = 2
NS = 16
C = 128
KB = 2
NCH = 160
NPH = NCH // KB
E_PAD = NS * NCH * C
N_PAD = 10240
RPT = N_PAD // NS
TPC = RPT // C

_f32 = jnp.float32
_mesh = plsc.VectorSubcoreMesh(core_axis_name="c", subcore_axis_name="s")
_sc_params = pltpu.CompilerParams(use_tc_tiling_on_sc=False)


def _hist_body(coli_hbm, z16_hbm, out_hbm, colidx_v, ones_v, acc_sh, *sems):
    c = lax.axis_index("c")
    s = lax.axis_index("s")
    w = c * NS + s
    pltpu.sync_copy(coli_hbm.at[w], colidx_v)

    def fill(i, carry):
        ones_v[i, :] = jnp.ones((16,), _f32)
        return carry

    lax.fori_loop(0, C, fill, 0)
    pltpu.sync_copy(z16_hbm.at[pl.ds(s * RPT, RPT)],
                    acc_sh.at[pl.ds(s * RPT, RPT)])
    plsc.subcore_barrier()

    def phase(p, carry):
        descs = [
            pltpu.async_copy(ones_v, acc_sh.at[colidx_v.at[p * KB + j]],
                             sems[j], add=True)
            for j in range(KB)
        ]
        for d in descs:
            d.wait()
        return carry

    lax.fori_loop(0, (NCH // NC) // KB, phase, 0)
    plsc.subcore_barrier()
    pltpu.sync_copy(acc_sh.at[pl.ds(s * RPT, RPT)],
                    out_hbm.at[c, pl.ds(s * RPT, RPT)])


_hist_kernel = pl.kernel(
    _hist_body,
    out_type=jax.ShapeDtypeStruct((NC, N_PAD, 16), _f32),
    mesh=_mesh,
    scratch_types=[
        pltpu.VMEM((NCH // NC, C), jnp.int32),
        pltpu.VMEM((C, 16), _f32),
        pltpu.VMEM_SHARED((N_PAD, 16), _f32),
    ] + [pltpu.SemaphoreType.DMA] * KB,
    compiler_params=_sc_params,
)


_RB = 1024
_GRID = N_PAD // _RB


def _norm_body(hist_ref, disb_ref, invcb_ref):
    cnt = hist_ref[0, :, 0:1] + hist_ref[1, :, 0:1]
    disb_ref[...] = jnp.broadcast_to(lax.rsqrt(cnt + 1.0), (_RB, 16))
    invcb_ref[...] = jnp.broadcast_to(1.0 / jnp.maximum(cnt, 1.0), (_RB, 16))


_norm_kernel = pl.pallas_call(
    _norm_body,
    grid=(_GRID,),
    in_specs=[pl.BlockSpec((NC, _RB, 16), lambda i: (0, i, 0))],
    out_specs=[pl.BlockSpec((_RB, 16), lambda i: (i, 0))] * 2,
    out_shape=[jax.ShapeDtypeStruct((N_PAD, 16), _f32)] * 2,
)


def _make_round(mode):
    def body(*refs):
        it = iter(refs)
        if mode == "prop1":
            x_hbm = next(it)
        else:
            g_hbm = next(it)
        rowi_hbm = next(it)
        coli_hbm = next(it)
        scb_hbm = next(it)
        if mode == "prop1":
            aux_hbm = x_hbm
        elif mode == "prop2":
            aux_hbm = next(it)
        else:
            aux_hbm = next(it)
        if mode == "prop1":
            g_hbm = next(it)
            out1_hbm = next(it)
            out2_hbm = next(it)
        elif mode == "prop2":
            out1_hbm = next(it)
        else:
            out1_hbm = next(it)
        rowidx_v = next(it)
        colidx_v = next(it)
        a0, a1, b0, b1 = next(it), next(it), next(it), next(it)
        scb_v = next(it)
        acc_sh = next(it)
        gsa, ssa, gsb, ssb = next(it), next(it), next(it), next(it)
        abufs = (a0, a1)
        bbufs = (b0, b1)

        c = lax.axis_index("c")
        s = lax.axis_index("s")
        pltpu.sync_copy(rowi_hbm.at[c, s], rowidx_v)
        pltpu.sync_copy(coli_hbm.at[s], colidx_v)
        pltpu.sync_copy(scb_hbm.at[pl.ds(s * RPT, RPT)], scb_v)

        if mode == "prop1":
            def initc(t, carry):
                lrow = s * RPT + t * C
                grow = c * N_PAD + lrow
                pltpu.sync_copy(x_hbm.at[pl.ds(grow, C)], a0)

                def rowf(r, carry2):
                    dv = scb_v[t * C + r, :]
                    for q in range(H // 16):
                        sl = pl.ds(q * 16, 16)
                        a0[r, sl] = dv * a0[r, sl]
                    return carry2

                lax.fori_loop(0, C, rowf, 0)
                pltpu.sync_copy(a0, g_hbm.at[pl.ds(grow, C)])
                pltpu.sync_copy(a0, acc_sh.at[pl.ds(lrow, C)])
                return carry

            lax.fori_loop(0, TPC, initc, 0)
        elif mode == "prop2":
            pltpu.sync_copy(g_hbm.at[pl.ds(c * N_PAD + s * RPT, RPT)],
                            acc_sh.at[pl.ds(s * RPT, RPT)])
        else:
            def zrow(r, carry):
                for q in range(H // 16):
                    a0[r, pl.ds(q * 16, 16)] = jnp.zeros((16,), _f32)
                return carry

            lax.fori_loop(0, C, zrow, 0)

            def zinit(t, carry):
                pltpu.sync_copy(a0, acc_sh.at[pl.ds(s * RPT + t * C, C)])
                return carry

            lax.fori_loop(0, TPC, zinit, 0)
        plsc.subcore_barrier()

        def gathers(p, bufs, sem):
            return [
                pltpu.async_copy(g_hbm.at[rowidx_v.at[p * KB + j]], bufs[j],
                                 sem)
                for j in range(KB)
            ]

        def scatters(p, bufs, sem):
            return [
                pltpu.async_copy(bufs[j], acc_sh.at[colidx_v.at[p * KB + j]],
                                 sem, add=True)
                for j in range(KB)
            ]

        def wait_all(descs):
            for d in descs:
                d.wait()

        wait_all(gathers(0, abufs, gsa))

        def pair(q, carry):
            pa = 2 * q
            pb = 2 * q + 1
            sa = scatters(pa, abufs, ssa)
            gb = gathers(pb, bbufs, gsb)
            wait_all(sa)
            wait_all(gb)
            sb = scatters(pb, bbufs, ssb)
            ga = gathers(pa + 2, abufs, gsa)
            wait_all(sb)
            wait_all(ga)
            return carry

        lax.fori_loop(0, NPH // 2 - 1, pair, 0)
        sa = scatters(NPH - 2, abufs, ssa)
        gb = gathers(NPH - 1, bbufs, gsb)
        wait_all(sa)
        wait_all(gb)
        wait_all(scatters(NPH - 1, bbufs, ssb))
        plsc.subcore_barrier()

        def outc(t, carry):
            lrow = s * RPT + t * C
            grow = c * N_PAD + lrow
            pltpu.sync_copy(acc_sh.at[pl.ds(lrow, C)], a0)
            pltpu.sync_copy(aux_hbm.at[pl.ds(grow, C)], b0)

            def rowf(r, carry2):
                dv = scb_v[t * C + r, :]
                for q in range(H // 16):
                    sl = pl.ds(q * 16, 16)
                    v = a0[r, sl]
                    xv = b0[r, sl]
                    if mode == "prop1":
                        h = dv * v
                        a0[r, sl] = dv * h
                        b0[r, sl] = ALPHA * xv + CK * h
                    elif mode == "prop2":
                        a0[r, sl] = jnp.maximum(xv + CK * (dv * v), 0.0)
                    else:
                        a0[r, sl] = jnp.maximum(dv * v + xv, 0.0)
                return carry2

            lax.fori_loop(0, C, rowf, 0)
            if mode == "prop1":
                pltpu.sync_copy(a0, out1_hbm.at[pl.ds(grow, C)])
                pltpu.sync_copy(b0, out2_hbm.at[pl.ds(grow, C)])
            elif mode == "prop2":
                pltpu.sync_copy(a0, out1_hbm.at[pl.ds(grow, C)])
            else:
                pltpu.sync_copy(a0, out1_hbm.at[c, pl.ds(lrow, C)])
            return carry

        lax.fori_loop(0, TPC, outc, 0)

    flat_shape = jax.ShapeDtypeStruct((NC * N_PAD, H), _f32)
    if mode == "prop1":
        out_type = [flat_shape, flat_shape, flat_shape]
    elif mode == "prop2":
        out_type = [flat_shape]
    else:
        out_type = [jax.ShapeDtypeStruct((NC, N_PAD, H), _f32)]

    return pl.kernel(
        body,
        out_type=out_type,
        mesh=_mesh,
        scratch_types=[
            pltpu.VMEM((NCH, C), jnp.int32),
            pltpu.VMEM((NCH, C), jnp.int32),
            pltpu.VMEM((C, H), _f32),
            pltpu.VMEM((C, H), _f32),
            pltpu.VMEM((C, H), _f32),
            pltpu.VMEM((C, H), _f32),
            pltpu.VMEM((RPT, 16), _f32),
            pltpu.VMEM_SHARED((N_PAD, H), _f32),
        ] + [pltpu.SemaphoreType.DMA] * 4,
        compiler_params=_sc_params,
    )


_prop1_kernel = _make_round("prop1")
_prop2_kernel = _make_round("prop2")
_mean_kernel = _make_round("mean")


@jax.jit
def kernel(x, edge_index):
    row = edge_index[0]
    col = edge_index[1]
    fill = (jnp.arange(E_PAD - E, dtype=jnp.int32) % (N_PAD - N)) + N
    row_flat = jnp.concatenate([row, fill])
    row_p = jnp.stack([row_flat, row_flat + N_PAD]).reshape(NC, NS, NCH, C)
    col_p = jnp.concatenate([col, fill]).reshape(NS, NCH, C)
    x_p = jnp.pad(x, ((0, N_PAD - N), (0, 0)))
    x2 = jnp.stack([x_p[:, :H], x_p[:, H:]]).reshape(NC * N_PAD, H)
    z16 = jnp.zeros((N_PAD, 16), _f32)

    hist = _hist_kernel(col_p.reshape(NC * NS, NCH // NC, C), z16)
    disb, invcb = _norm_kernel(hist)
    _, g1, o1 = _prop1_kernel(x2, row_p, col_p, disb)
    (r,) = _prop2_kernel(g1, row_p, col_p, disb, o1)
    (fin,) = _mean_kernel(r, row_p, col_p, invcb, x2)
    return jnp.concatenate([fin[0, :N], fin[1, :N]], axis=1)


# speedup vs baseline: 22.1294x; 1.0803x over previous
import jax
import jax.numpy as jnp
from jax import lax
from jax.experimental import pallas as pl
from jax.experimental.pallas import tpu as pltpu
from jax.experimental.pallas import tpu_sc as plsc

N = 10000
D = 128
H = D // 2
E = 320000
ALPHA = 0.1
CK = 0.45

NC = 2
NS = 16
C = 128
KB = 2
KR = 4
NCH = 160
NPH = NCH // KR
E_PAD = NS * NCH * C
N_PAD = 10240
RPT = N_PAD // NS
TPC = RPT // C

_f32 = jnp.float32
_mesh = plsc.VectorSubcoreMesh(core_axis_name="c", subcore_axis_name="s")
_sc_params = pltpu.CompilerParams(use_tc_tiling_on_sc=False)


def _hist_body(coli_hbm, z16_hbm, out_hbm, colidx_v, ones_v, acc_sh, *sems):
    c = lax.axis_index("c")
    s = lax.axis_index("s")
    w = c * NS + s
    pltpu.sync_copy(coli_hbm.at[w], colidx_v)

    def fill(i, carry):
        ones_v[i, :] = jnp.ones((16,), _f32)
        return carry

    lax.fori_loop(0, C, fill, 0)
    pltpu.sync_copy(z16_hbm.at[pl.ds(s * RPT, RPT)],
                    acc_sh.at[pl.ds(s * RPT, RPT)])
    plsc.subcore_barrier()

    def phase(p, carry):
        descs = [
            pltpu.async_copy(ones_v, acc_sh.at[colidx_v.at[p * KB + j]],
                             sems[j], add=True)
            for j in range(KB)
        ]
        for d in descs:
            d.wait()
        return carry

    lax.fori_loop(0, (NCH // NC) // KB, phase, 0)
    plsc.subcore_barrier()
    pltpu.sync_copy(acc_sh.at[pl.ds(s * RPT, RPT)],
                    out_hbm.at[c, pl.ds(s * RPT, RPT)])


_hist_kernel = pl.kernel(
    _hist_body,
    out_type=jax.ShapeDtypeStruct((NC, N_PAD, 16), _f32),
    mesh=_mesh,
    scratch_types=[
        pltpu.VMEM((NCH // NC, C), jnp.int32),
        pltpu.VMEM((C, 16), _f32),
        pltpu.VMEM_SHARED((N_PAD, 16), _f32),
    ] + [pltpu.SemaphoreType.DMA] * KB,
    compiler_params=_sc_params,
)


_RB = 1024
_GRID = N_PAD // _RB


def _norm_body(hist_ref, disb_ref, invcb_ref):
    cnt = hist_ref[0, :, 0:1] + hist_ref[1, :, 0:1]
    disb_ref[...] = jnp.broadcast_to(lax.rsqrt(cnt + 1.0), (_RB, 16))
    invcb_ref[...] = jnp.broadcast_to(1.0 / jnp.maximum(cnt, 1.0), (_RB, 16))


_norm_kernel = pl.pallas_call(
    _norm_body,
    grid=(_GRID,),
    in_specs=[pl.BlockSpec((NC, _RB, 16), lambda i: (0, i, 0))],
    out_specs=[pl.BlockSpec((_RB, 16), lambda i: (i, 0))] * 2,
    out_shape=[jax.ShapeDtypeStruct((N_PAD, 16), _f32)] * 2,
)


def _make_round(mode):
    def body(*refs):
        it = iter(refs)
        if mode == "prop1":
            x_hbm = next(it)
        else:
            g_hbm = next(it)
        rc_hbm = next(it)
        scb_hbm = next(it)
        if mode == "prop1":
            aux_hbm = x_hbm
        elif mode == "prop2":
            aux_hbm = next(it)
        else:
            aux_hbm = next(it)
        if mode == "prop1":
            g_hbm = next(it)
            out1_hbm = next(it)
            out2_hbm = next(it)
        elif mode == "prop2":
            out1_hbm = next(it)
        else:
            out1_hbm = next(it)
        ia_v = next(it)
        ib_v = next(it)
        abufs = tuple(next(it) for _ in range(KR))
        bbufs = tuple(next(it) for _ in range(KR))
        a0 = abufs[0]
        b0 = bbufs[0]
        scb_v = next(it)
        acc_sh = next(it)
        gsa, ssa, gsb, ssb, isa, isb = (next(it) for _ in range(6))

        c = lax.axis_index("c")
        s = lax.axis_index("s")
        pltpu.sync_copy(scb_hbm.at[pl.ds(s * RPT, RPT)], scb_v)

        if mode == "prop1":
            def initc(t, carry):
                lrow = s * RPT + t * C
                grow = c * N_PAD + lrow
                pltpu.sync_copy(x_hbm.at[pl.ds(grow, C)], a0)

                def rowf(r, carry2):
                    dv = scb_v[t * C + r, :]
                    for q in range(H // 16):
                        sl = pl.ds(q * 16, 16)
                        a0[r, sl] = dv * a0[r, sl]
                    return carry2

                lax.fori_loop(0, C, rowf, 0)
                pltpu.sync_copy(a0, g_hbm.at[pl.ds(grow, C)])
                pltpu.sync_copy(a0, acc_sh.at[pl.ds(lrow, C)])
                return carry

            lax.fori_loop(0, TPC, initc, 0)
        elif mode == "prop2":
            pltpu.sync_copy(g_hbm.at[pl.ds(c * N_PAD + s * RPT, RPT)],
                            acc_sh.at[pl.ds(s * RPT, RPT)])
        else:
            def zrow(r, carry):
                for q in range(H // 16):
                    a0[r, pl.ds(q * 16, 16)] = jnp.zeros((16,), _f32)
                return carry

            lax.fori_loop(0, C, zrow, 0)

            def zinit(t, carry):
                pltpu.sync_copy(a0, acc_sh.at[pl.ds(s * RPT + t * C, C)])
                return carry

            lax.fori_loop(0, TPC, zinit, 0)
        plsc.subcore_barrier()

        def gathers(bufs, idxv, sem):
            return [
                pltpu.async_copy(g_hbm.at[idxv.at[j]], bufs[j], sem)
                for j in range(KR)
            ]

        def scatters(bufs, idxv, sem):
            return [
                pltpu.async_copy(bufs[j], acc_sh.at[idxv.at[KR + j]],
                                 sem, add=True)
                for j in range(KR)
            ]

        def prefetch(p, idxv, sem):
            return pltpu.async_copy(rc_hbm.at[c, s, p], idxv, sem)

        def wait_all(descs):
            for d in descs:
                d.wait()

        pltpu.sync_copy(rc_hbm.at[c, s, 0], ia_v)
        wait_all(gathers(abufs, ia_v, gsa))
        pltpu.sync_copy(rc_hbm.at[c, s, 1], ib_v)

        def pair(q, carry):
            pa = 2 * q
            pb = 2 * q + 1
            sa = scatters(abufs, ia_v, ssa)
            gb = gathers(bbufs, ib_v, gsb)
            wait_all(sa)
            ia = prefetch(pa + 2, ia_v, isa)
            wait_all(gb)
            sb = scatters(bbufs, ib_v, ssb)
            ia.wait()
            ga = gathers(abufs, ia_v, gsa)
            wait_all(sb)
            ib = prefetch(pb + 2, ib_v, isb)
            wait_all(ga)
            ib.wait()
            return carry

        lax.fori_loop(0, NPH // 2 - 1, pair, 0)
        sa = scatters(abufs, ia_v, ssa)
        gb = gathers(bbufs, ib_v, gsb)
        wait_all(sa)
        wait_all(gb)
        wait_all(scatters(bbufs, ib_v, ssb))
        plsc.subcore_barrier()

        def outc(t, carry):
            lrow = s * RPT + t * C
            grow = c * N_PAD + lrow
            pltpu.sync_copy(acc_sh.at[pl.ds(lrow, C)], a0)
            pltpu.sync_copy(aux_hbm.at[pl.ds(grow, C)], b0)

            def rowf(r, carry2):
                dv = scb_v[t * C + r, :]
                for q in range(H // 16):
                    sl = pl.ds(q * 16, 16)
                    v = a0[r, sl]
                    xv = b0[r, sl]
                    if mode == "prop1":
                        h = dv * v
                        a0[r, sl] = dv * h
                        b0[r, sl] = ALPHA * xv + CK * h
                    elif mode == "prop2":
                        a0[r, sl] = jnp.maximum(xv + CK * (dv * v), 0.0)
                    else:
                        a0[r, sl] = jnp.maximum(dv * v + xv, 0.0)
                return carry2

            lax.fori_loop(0, C, rowf, 0)
            if mode == "prop1":
                pltpu.sync_copy(a0, out1_hbm.at[pl.ds(grow, C)])
                pltpu.sync_copy(b0, out2_hbm.at[pl.ds(grow, C)])
            elif mode == "prop2":
                pltpu.sync_copy(a0, out1_hbm.at[pl.ds(grow, C)])
            else:
                pltpu.sync_copy(a0, out1_hbm.at[c, pl.ds(lrow, C)])
            return carry

        lax.fori_loop(0, TPC, outc, 0)

    flat_shape = jax.ShapeDtypeStruct((NC * N_PAD, H), _f32)
    if mode == "prop1":
        out_type = [flat_shape, flat_shape, flat_shape]
    elif mode == "prop2":
        out_type = [flat_shape]
    else:
        out_type = [jax.ShapeDtypeStruct((NC, N_PAD, H), _f32)]

    return pl.kernel(
        body,
        out_type=out_type,
        mesh=_mesh,
        scratch_types=[
            pltpu.VMEM((2 * KR, C), jnp.int32),
            pltpu.VMEM((2 * KR, C), jnp.int32),
        ] + [pltpu.VMEM((C, H), _f32)] * (2 * KR) + [
            pltpu.VMEM((RPT, 16), _f32),
            pltpu.VMEM_SHARED((N_PAD, H), _f32),
        ] + [pltpu.SemaphoreType.DMA] * 6,
        compiler_params=_sc_params,
    )


_prop1_kernel = _make_round("prop1")
_prop2_kernel = _make_round("prop2")
_mean_kernel = _make_round("mean")


@jax.jit
def kernel(x, edge_index):
    row = edge_index[0]
    col = edge_index[1]
    fill = (jnp.arange(E_PAD - E, dtype=jnp.int32) % (N_PAD - N)) + N
    row_flat = jnp.concatenate([row, fill])
    col_flat = jnp.concatenate([col, fill])
    row_p = jnp.stack([row_flat, row_flat + N_PAD]).reshape(NC, NS, NPH, KR, C)
    col_p = jnp.broadcast_to(
        col_flat.reshape(1, NS, NPH, KR, C), (NC, NS, NPH, KR, C))
    rc_p = jnp.concatenate([row_p, col_p], axis=3)
    x_p = jnp.pad(x, ((0, N_PAD - N), (0, 0)))
    x2 = jnp.stack([x_p[:, :H], x_p[:, H:]]).reshape(NC * N_PAD, H)
    z16 = jnp.zeros((N_PAD, 16), _f32)

    hist = _hist_kernel(col_flat.reshape(NC * NS, NCH // NC, C), z16)
    disb, invcb = _norm_kernel(hist)
    _, g1, o1 = _prop1_kernel(x2, rc_p, disb)
    (r,) = _prop2_kernel(g1, rc_p, disb, o1)
    (fin,) = _mean_kernel(r, rc_p, invcb, x2)
    return jnp.concatenate([fin[0, :N], fin[1, :N]], axis=1)
